# Initial kernel scaffold; baseline (speedup 1.0000x reference)
import functools

import jax
import jax.numpy as jnp
from jax import lax
from jax.experimental import pallas as pl
from jax.experimental.pallas import tpu as pltpu
from jax.experimental.pallas import tpu_sc as plsc

V = 100000
D = 64
B = 1024
C = 20

NC = 2
NS = 16
NW = NC * NS
EPW = B // NW
CPW = EPW * C
ICH = 128
NCH = CPW // ICH

TV = 2048
NVT = (V + TV - 1) // TV


def _sc_gather(ctx2, in_tab, out_tab, bias16, lab_ids, row_ids):
  mesh = plsc.VectorSubcoreMesh(core_axis_name="c", subcore_axis_name="s")

  @functools.partial(
      pl.kernel,
      out_type=(
          jax.ShapeDtypeStruct((B, D), jnp.float32),
          jax.ShapeDtypeStruct((B, D), jnp.float32),
          jax.ShapeDtypeStruct((B, 16), jnp.float32),
      ),
      mesh=mesh,
      scratch_types=[
          pltpu.VMEM((NCH, ICH), jnp.int32),
          pltpu.VMEM((CPW, D), jnp.float32),
          pltpu.VMEM((EPW, D), jnp.float32),
          pltpu.VMEM((EPW,), jnp.int32),
          pltpu.VMEM((EPW, D), jnp.float32),
          pltpu.VMEM((EPW,), jnp.int32),
          pltpu.VMEM((EPW, 16), jnp.float32),
          pltpu.SemaphoreType.DMA,
          pltpu.SemaphoreType.DMA,
      ],
  )
  def k(ctx_h, itab_h, otab_h, b16_h, lab_h, rid_h, x_h, le_h, br_h,
        idxc, rows, xout, idxl, labv, ridv, brv, sem, sem2):
    wid = lax.axis_index("s") * NC + lax.axis_index("c")
    eb = wid * EPW

    pltpu.sync_copy(ctx_h.at[pl.ds(wid * NCH, NCH)], idxc)
    hs = [
        pltpu.async_copy(itab_h.at[idxc.at[j]],
                         rows.at[pl.ds(j * ICH, ICH)], sem)
        for j in range(NCH)
    ]

    pltpu.sync_copy(lab_h.at[pl.ds(eb, EPW)], idxl)
    pltpu.sync_copy(rid_h.at[pl.ds(eb, EPW)], ridv)
    h2 = pltpu.async_copy(otab_h.at[idxl], labv, sem2)
    h3 = pltpu.async_copy(b16_h.at[ridv], brv, sem2)

    for h in hs:
      h.wait()

    def esum(e, carry):
      for d in range(D // 16):
        sl = pl.ds(d * 16, 16)
        acc = rows[e * C, sl]
        for c2 in range(1, C):
          acc = acc + rows[e * C + c2, sl]
        xout[e, sl] = acc
      return carry

    lax.fori_loop(0, EPW, esum, 0)
    pltpu.sync_copy(xout, x_h.at[pl.ds(eb, EPW)])

    h2.wait()
    h3.wait()
    pltpu.sync_copy(labv, le_h.at[pl.ds(eb, EPW)])
    pltpu.sync_copy(brv, br_h.at[pl.ds(eb, EPW)])

  return k(ctx2, in_tab, out_tab, bias16, lab_ids, row_ids)


def _tc_body(x_ref, w_ref, b_ref, le_ref, br_ref, id_ref, o_ref, m_ref, s_ref):
  i = pl.program_id(0)
  nv = pl.num_programs(0)

  @pl.when(i == 0)
  def _():
    m_ref[...] = jnp.full((B, 1), -1e30, jnp.float32)
    s_ref[...] = jnp.zeros((B, 1), jnp.float32)

  x = x_ref[...]
  w = w_ref[...]
  logits = lax.dot_general(x, w, (((1,), (1,)), ((), ())),
                           preferred_element_type=jnp.float32) + b_ref[...]

  def _mask(l):
    col = i * TV + lax.broadcasted_iota(jnp.int32, (B, TV), 1)
    return jnp.where(col < V, l, -1e30)

  logits = lax.cond(i == nv - 1, _mask, lambda l: l, logits)

  tmax = jnp.max(logits, axis=1, keepdims=True)
  m_old = m_ref[...]
  m_new = jnp.maximum(m_old, tmax)
  p = jnp.sum(jnp.exp(logits - m_new), axis=1, keepdims=True)
  s_ref[...] = s_ref[...] * jnp.exp(m_old - m_new) + p
  m_ref[...] = m_new

  @pl.when(i == nv - 1)
  def _():
    lab_logit = jnp.sum(x * le_ref[...], axis=1, keepdims=True)
    lane = lax.rem(id_ref[...], 16)
    sel = lax.broadcasted_iota(jnp.int32, (B, 16), 1) == lane
    lbias = jnp.sum(jnp.where(sel, br_ref[...], 0.0), axis=1, keepdims=True)
    o_ref[...] = m_ref[...] + jnp.log(s_ref[...]) - lab_logit - lbias


def _tc_loss(x, w, bias2, labemb, brows, ids2):
  return pl.pallas_call(
      _tc_body,
      grid=(NVT,),
      in_specs=[
          pl.BlockSpec((B, D), lambda i: (0, 0)),
          pl.BlockSpec((TV, D), lambda i: (i, 0)),
          pl.BlockSpec((1, TV), lambda i: (0, i)),
          pl.BlockSpec((B, D), lambda i: (0, 0)),
          pl.BlockSpec((B, 16), lambda i: (0, 0)),
          pl.BlockSpec((B, 1), lambda i: (0, 0)),
      ],
      out_specs=pl.BlockSpec((B, 1), lambda i: (0, 0)),
      out_shape=jax.ShapeDtypeStruct((B, 1), jnp.float32),
      scratch_shapes=[
          pltpu.VMEM((B, 1), jnp.float32),
          pltpu.VMEM((B, 1), jnp.float32),
      ],
  )(x, w, bias2, labemb, brows, ids2)


def kernel(input_word_ids, output_word_ids, input_layer_embeddings,
           output_layer_embeddings, output_layer_bias):
  ctx2 = input_word_ids.reshape(B * C // ICH, ICH)
  bias16 = output_layer_bias.reshape(V // 16, 16)
  row_ids = lax.shift_right_logical(output_word_ids, 4)

  x, labemb, brows = _sc_gather(ctx2, input_layer_embeddings,
                                output_layer_embeddings, bias16,
                                output_word_ids, row_ids)

  bias2 = output_layer_bias.reshape(1, V)
  ids2 = output_word_ids.reshape(B, 1)
  loss = _tc_loss(x, output_layer_embeddings, bias2, labemb, brows, ids2)
  return loss.reshape(B)


# speedup vs baseline: 1.0931x; 1.0931x over previous
import functools

import jax
import jax.numpy as jnp
from jax import lax
from jax.experimental import pallas as pl
from jax.experimental.pallas import tpu as pltpu
from jax.experimental.pallas import tpu_sc as plsc

V = 100000
D = 64
B = 1024
C = 20

NC = 2
NS = 16
NW = NC * NS
EPW = B // NW
CPW = EPW * C
ICH = 128
NCH = CPW // ICH

TV = 2048
NVT = (V + TV - 1) // TV


def _sc_gather(ctx2, in_tab, out_tab, bias16, lab_ids, row_ids):
  mesh = plsc.VectorSubcoreMesh(core_axis_name="c", subcore_axis_name="s")

  @functools.partial(
      pl.kernel,
      out_type=(
          jax.ShapeDtypeStruct((B, D), jnp.float32),
          jax.ShapeDtypeStruct((B, D), jnp.float32),
          jax.ShapeDtypeStruct((B, 16), jnp.float32),
      ),
      mesh=mesh,
      compiler_params=pltpu.CompilerParams(use_tc_tiling_on_sc=False),
      scratch_types=[
          pltpu.VMEM((CPW,), jnp.int32),
          pltpu.VMEM((CPW, D), jnp.float32),
          pltpu.VMEM((EPW, D), jnp.float32),
          pltpu.VMEM((EPW,), jnp.int32),
          pltpu.VMEM((EPW, D), jnp.float32),
          pltpu.VMEM((EPW,), jnp.int32),
          pltpu.VMEM((EPW, 16), jnp.float32),
          pltpu.SemaphoreType.DMA,
          pltpu.SemaphoreType.DMA,
      ],
  )
  def k(ctx_h, itab_h, otab_h, b16_h, lab_h, rid_h, x_h, le_h, br_h,
        idxc, rows, xout, idxl, labv, ridv, brv, sem, sem2):
    wid = lax.axis_index("s") * NC + lax.axis_index("c")
    eb = wid * EPW

    pltpu.sync_copy(ctx_h.at[pl.ds(wid * CPW, CPW)], idxc)
    hs = [
        pltpu.async_copy(itab_h.at[idxc.at[pl.ds(j * ICH, ICH)]],
                         rows.at[pl.ds(j * ICH, ICH)], sem)
        for j in range(NCH)
    ]

    pltpu.sync_copy(lab_h.at[pl.ds(eb, EPW)], idxl)
    pltpu.sync_copy(rid_h.at[pl.ds(eb, EPW)], ridv)
    h2 = pltpu.async_copy(otab_h.at[idxl], labv, sem2)
    h3 = pltpu.async_copy(b16_h.at[ridv], brv, sem2)

    for h in hs:
      h.wait()

    def esum(e, carry):
      for d in range(D // 16):
        sl = pl.ds(d * 16, 16)
        acc = rows[e * C, sl]
        for c2 in range(1, C):
          acc = acc + rows[e * C + c2, sl]
        xout[e, sl] = acc
      return carry

    lax.fori_loop(0, EPW, esum, 0)
    pltpu.sync_copy(xout, x_h.at[pl.ds(eb, EPW)])

    h2.wait()
    h3.wait()
    pltpu.sync_copy(labv, le_h.at[pl.ds(eb, EPW)])
    pltpu.sync_copy(brv, br_h.at[pl.ds(eb, EPW)])

  return k(ctx2, in_tab, out_tab, bias16, lab_ids, row_ids)


def _tc_body(x_ref, w_ref, b_ref, le_ref, br_ref, id_ref, o_ref, m_ref, s_ref):
  i = pl.program_id(0)
  nv = pl.num_programs(0)

  @pl.when(i == 0)
  def _():
    m_ref[...] = jnp.full((B, 1), -1e30, jnp.float32)
    s_ref[...] = jnp.zeros((B, 1), jnp.float32)

  x = x_ref[...]
  w = w_ref[...]
  logits = lax.dot_general(x, w, (((1,), (1,)), ((), ())),
                           preferred_element_type=jnp.float32) + b_ref[...]

  def _mask(l):
    col = i * TV + lax.broadcasted_iota(jnp.int32, (B, TV), 1)
    return jnp.where(col < V, l, -1e30)

  logits = lax.cond(i == nv - 1, _mask, lambda l: l, logits)

  tmax = jnp.max(logits, axis=1, keepdims=True)
  m_old = m_ref[...]
  m_new = jnp.maximum(m_old, tmax)
  p = jnp.sum(jnp.exp(logits - m_new), axis=1, keepdims=True)
  s_ref[...] = s_ref[...] * jnp.exp(m_old - m_new) + p
  m_ref[...] = m_new

  @pl.when(i == nv - 1)
  def _():
    lab_logit = jnp.sum(x * le_ref[...], axis=1, keepdims=True)
    lane = lax.rem(id_ref[...], 16)
    sel = lax.broadcasted_iota(jnp.int32, (B, 16), 1) == lane
    lbias = jnp.sum(jnp.where(sel, br_ref[...], 0.0), axis=1, keepdims=True)
    o_ref[...] = m_ref[...] + jnp.log(s_ref[...]) - lab_logit - lbias


def _tc_loss(x, w, bias2, labemb, brows, ids2):
  return pl.pallas_call(
      _tc_body,
      grid=(NVT,),
      in_specs=[
          pl.BlockSpec((B, D), lambda i: (0, 0)),
          pl.BlockSpec((TV, D), lambda i: (i, 0)),
          pl.BlockSpec((1, TV), lambda i: (0, i)),
          pl.BlockSpec((B, D), lambda i: (0, 0)),
          pl.BlockSpec((B, 16), lambda i: (0, 0)),
          pl.BlockSpec((B, 1), lambda i: (0, 0)),
      ],
      out_specs=pl.BlockSpec((B, 1), lambda i: (0, 0)),
      out_shape=jax.ShapeDtypeStruct((B, 1), jnp.float32),
      scratch_shapes=[
          pltpu.VMEM((B, 1), jnp.float32),
          pltpu.VMEM((B, 1), jnp.float32),
      ],
  )(x, w, bias2, labemb, brows, ids2)


def kernel(input_word_ids, output_word_ids, input_layer_embeddings,
           output_layer_embeddings, output_layer_bias):
  ctx2 = input_word_ids.reshape(B * C)
  bias16 = output_layer_bias.reshape(V // 16, 16)
  row_ids = lax.shift_right_logical(output_word_ids, 4)

  x, labemb, brows = _sc_gather(ctx2, input_layer_embeddings,
                                output_layer_embeddings, bias16,
                                output_word_ids, row_ids)

  bias2 = output_layer_bias.reshape(1, V)
  ids2 = output_word_ids.reshape(B, 1)
  loss = _tc_loss(x, output_layer_embeddings, bias2, labemb, brows, ids2)
  return loss.reshape(B)


# speedup vs baseline: 1.1555x; 1.0571x over previous
import functools

import jax
import jax.numpy as jnp
from jax import lax
from jax.experimental import pallas as pl
from jax.experimental.pallas import tpu as pltpu
from jax.experimental.pallas import tpu_sc as plsc

V = 100000
D = 64
B = 1024
C = 20

NC = 2
NS = 16
NW = NC * NS
EPW = B // NW
CPW = EPW * C
ICH = 128
NCH = CPW // ICH

KA = 128
TV = 2048
VP = ((V + TV - 1) // TV) * TV
NVT = VP // TV

_SC_PARAMS = pltpu.CompilerParams(use_tc_tiling_on_sc=False)


def _sc_ctx_gather(ctx_flat, in_tab):
  mesh = plsc.VectorSubcoreMesh(core_axis_name="c", subcore_axis_name="s")

  @functools.partial(
      pl.kernel,
      out_type=jax.ShapeDtypeStruct((B, D), jnp.float32),
      mesh=mesh,
      compiler_params=_SC_PARAMS,
      scratch_types=[
          pltpu.VMEM((CPW,), jnp.int32),
          pltpu.VMEM((CPW, D), jnp.float32),
          pltpu.VMEM((EPW, D), jnp.float32),
          pltpu.SemaphoreType.DMA,
      ],
  )
  def k(ctx_h, itab_h, x_h, idxc, rows, xout, sem):
    wid = lax.axis_index("s") * NC + lax.axis_index("c")
    eb = wid * EPW

    pltpu.sync_copy(ctx_h.at[pl.ds(wid * CPW, CPW)], idxc)
    hs = [
        pltpu.async_copy(itab_h.at[idxc.at[pl.ds(j * ICH, ICH)]],
                         rows.at[pl.ds(j * ICH, ICH)], sem)
        for j in range(NCH)
    ]
    for h in hs:
      h.wait()

    def esum(e, carry):
      for d in range(D // 16):
        sl = pl.ds(d * 16, 16)
        acc = rows[e * C, sl]
        for c2 in range(1, C):
          acc = acc + rows[e * C + c2, sl]
        xout[e, sl] = acc
      return carry

    lax.fori_loop(0, EPW, esum, 0)
    pltpu.sync_copy(xout, x_h.at[pl.ds(eb, EPW)])

  return k(ctx_flat, in_tab)


def _sc_label_gather(out_tab, bias16, lab_ids, row_ids):
  mesh = plsc.VectorSubcoreMesh(core_axis_name="c", subcore_axis_name="s")

  @functools.partial(
      pl.kernel,
      out_type=(
          jax.ShapeDtypeStruct((B, D), jnp.float32),
          jax.ShapeDtypeStruct((B, 16), jnp.float32),
      ),
      mesh=mesh,
      compiler_params=_SC_PARAMS,
      scratch_types=[
          pltpu.VMEM((EPW,), jnp.int32),
          pltpu.VMEM((EPW, D), jnp.float32),
          pltpu.VMEM((EPW,), jnp.int32),
          pltpu.VMEM((EPW, 16), jnp.float32),
          pltpu.SemaphoreType.DMA,
      ],
  )
  def k(otab_h, b16_h, lab_h, rid_h, le_h, br_h, idxl, labv, ridv, brv, sem):
    wid = lax.axis_index("s") * NC + lax.axis_index("c")
    eb = wid * EPW
    pltpu.sync_copy(lab_h.at[pl.ds(eb, EPW)], idxl)
    pltpu.sync_copy(rid_h.at[pl.ds(eb, EPW)], ridv)
    h2 = pltpu.async_copy(otab_h.at[idxl], labv, sem)
    h3 = pltpu.async_copy(b16_h.at[ridv], brv, sem)
    h2.wait()
    h3.wait()
    pltpu.sync_copy(labv, le_h.at[pl.ds(eb, EPW)])
    pltpu.sync_copy(brv, br_h.at[pl.ds(eb, EPW)])

  return k(out_tab, bias16, lab_ids, row_ids)


def _lse_body(x_ref, w_ref, o_ref, m_ref, s_ref):
  i = pl.program_id(0)
  nv = pl.num_programs(0)

  @pl.when(i == 0)
  def _():
    m_ref[...] = jnp.full((B, 1), -1e30, jnp.float32)
    s_ref[...] = jnp.zeros((B, 1), jnp.float32)

  x = x_ref[...]
  w = w_ref[...]
  t = lax.dot_general(x, w, (((1,), (1,)), ((), ())),
                      preferred_element_type=jnp.float32).astype(jnp.bfloat16)
  tmax = jnp.max(t, axis=1, keepdims=True).astype(jnp.float32)
  m_old = m_ref[...]
  m_new = jnp.maximum(m_old, tmax)
  mb = m_new.astype(jnp.bfloat16)
  e = jnp.exp(t - mb)
  ones = jnp.ones((TV, 1), jnp.bfloat16)
  p = lax.dot_general(e, ones, (((1,), (0,)), ((), ())),
                      preferred_element_type=jnp.float32)
  s_ref[...] = s_ref[...] * jnp.exp(m_old - m_new) + p
  m_ref[...] = m_new

  @pl.when(i == nv - 1)
  def _():
    o_ref[...] = m_ref[...] + jnp.log(s_ref[...])


def _tc_lse(x_aug, w_aug):
  return pl.pallas_call(
      _lse_body,
      grid=(NVT,),
      in_specs=[
          pl.BlockSpec((B, KA), lambda i: (0, 0)),
          pl.BlockSpec((TV, KA), lambda i: (i, 0)),
      ],
      out_specs=pl.BlockSpec((B, 1), lambda i: (0, 0)),
      out_shape=jax.ShapeDtypeStruct((B, 1), jnp.float32),
      scratch_shapes=[
          pltpu.VMEM((B, 1), jnp.float32),
          pltpu.VMEM((B, 1), jnp.float32),
      ],
  )(x_aug, w_aug)


def _final_body(lse_ref, x_ref, le_ref, br_ref, id_ref, o_ref):
  lab_logit = jnp.sum(x_ref[...] * le_ref[...], axis=1, keepdims=True)
  lane = lax.rem(id_ref[...], 16)
  sel = lax.broadcasted_iota(jnp.int32, (B, 16), 1) == lane
  lbias = jnp.sum(jnp.where(sel, br_ref[...], 0.0), axis=1, keepdims=True)
  o_ref[...] = lse_ref[...] - lab_logit - lbias


def _tc_final(lse, x, labemb, brows, ids2):
  return pl.pallas_call(
      _final_body,
      out_shape=jax.ShapeDtypeStruct((B, 1), jnp.float32),
  )(lse, x, labemb, brows, ids2)


def kernel(input_word_ids, output_word_ids, input_layer_embeddings,
           output_layer_embeddings, output_layer_bias):
  ctx_flat = input_word_ids.reshape(B * C)
  bias16 = output_layer_bias.reshape(V // 16, 16)
  row_ids = lax.shift_right_logical(output_word_ids, 4)

  x = _sc_ctx_gather(ctx_flat, input_layer_embeddings)
  labemb, brows = _sc_label_gather(output_layer_embeddings, bias16,
                                   output_word_ids, row_ids)

  w_pad = jnp.pad(output_layer_embeddings, ((0, VP - V), (0, 0)))
  bias_pad = jnp.pad(output_layer_bias, (0, VP - V), constant_values=-1e30)
  w_aug = jnp.concatenate(
      [w_pad, bias_pad[:, None],
       jnp.zeros((VP, KA - D - 1), jnp.float32)], axis=1).astype(jnp.bfloat16)
  x_aug = jnp.concatenate(
      [x, jnp.ones((B, 1), jnp.float32),
       jnp.zeros((B, KA - D - 1), jnp.float32)], axis=1).astype(jnp.bfloat16)

  lse = _tc_lse(x_aug, w_aug)
  ids2 = output_word_ids.reshape(B, 1)
  loss = _tc_final(lse, x, labemb, brows, ids2)
  return loss.reshape(B)


# speedup vs baseline: 1.3589x; 1.1760x over previous
import functools

import jax
import jax.numpy as jnp
from jax import lax
from jax.experimental import pallas as pl
from jax.experimental.pallas import tpu as pltpu
from jax.experimental.pallas import tpu_sc as plsc

V = 100000
D = 64
B = 1024
C = 20

NC = 2
NS = 16
NW = NC * NS
EPW = B // NW
CPW = EPW * C
ICH = 128
NCH = CPW // ICH

VH = V // 2
TP = 1024
NPT = (VH + TP - 1) // TP
HP = NPT * TP
BT = 2 * HP // 128

_SC_PARAMS = pltpu.CompilerParams(use_tc_tiling_on_sc=True,
                                  needs_layout_passes=False)
NEG = -1e30


def _prepw_body(w_ref, o_ref):
  i = pl.program_id(0)
  nv = pl.num_programs(0)
  w = w_ref[...]

  def _mask(wv):
    row = i * TP + lax.broadcasted_iota(jnp.int32, (TP, 1), 0)
    return jnp.where(row < VH, wv, 0.0)

  w = lax.cond(i == nv - 1, _mask, lambda wv: wv, w)
  o_ref[...] = w.astype(jnp.bfloat16)


def _prep_w(out_pair):
  return pl.pallas_call(
      _prepw_body,
      grid=(NPT,),
      in_specs=[pl.BlockSpec((TP, 128), lambda i: (i, 0))],
      out_specs=pl.BlockSpec((TP, 128), lambda i: (i, 0)),
      out_shape=jax.ShapeDtypeStruct((HP, 128), jnp.bfloat16),
  )(out_pair)


def _sc_ctx_gather(ctx_prow, ctx_par, in_pair):
  mesh = plsc.VectorSubcoreMesh(core_axis_name="c", subcore_axis_name="s")

  @functools.partial(
      pl.kernel,
      out_type=jax.ShapeDtypeStruct((B, 128), jnp.float32),
      mesh=mesh,
      compiler_params=_SC_PARAMS,
      scratch_types=[
          pltpu.VMEM((CPW,), jnp.int32),
          pltpu.VMEM((CPW,), jnp.int32),
          pltpu.VMEM((CPW, 128), jnp.float32),
          pltpu.VMEM((EPW, 128), jnp.float32),
          pltpu.SemaphoreType.DMA,
      ],
  )
  def k(prow_h, par_h, pair_h, x_h, idxc, parv, rows, xout, sem):
    wid = lax.axis_index("s") * NC + lax.axis_index("c")
    eb = wid * EPW

    pltpu.sync_copy(prow_h.at[pl.ds(wid * CPW, CPW)], idxc)
    pltpu.sync_copy(par_h.at[pl.ds(wid * CPW, CPW)], parv)
    hs = [
        pltpu.async_copy(pair_h.at[idxc.at[pl.ds(j * ICH, ICH)]],
                         rows.at[pl.ds(j * ICH, ICH)], sem)
        for j in range(NCH)
    ]
    for h in hs:
      h.wait()

    one16 = jnp.where(lax.iota(jnp.int32, 16) == 0, 1.0, 0.0)
    z16 = jnp.zeros((16,), jnp.float32)

    def esum(e, carry):
      masks = []
      for c2 in range(C):
        psplat = plsc.load_gather(parv, [jnp.full((16,), e * C + c2,
                                                  jnp.int32)])
        masks.append(psplat == 1)
      accs = []
      for d in range(D // 16):
        r = e * C
        acc = jnp.where(masks[0], rows[r, pl.ds(D + d * 16, 16)],
                        rows[r, pl.ds(d * 16, 16)])
        for c2 in range(1, C):
          r = e * C + c2
          acc = acc + jnp.where(masks[c2], rows[r, pl.ds(D + d * 16, 16)],
                                rows[r, pl.ds(d * 16, 16)])
        accs.append(acc)
      for d in range(D // 16):
        xout[e, pl.ds(d * 16, 16)] = accs[d]
      xout[e, pl.ds(D, 16)] = one16
      for d in range(D // 16 + 1, 128 // 16):
        xout[e, pl.ds(d * 16, 16)] = z16
      return carry

    lax.fori_loop(0, EPW, esum, 0)
    pltpu.sync_copy(xout, x_h.at[pl.ds(eb, EPW)])

  return k(ctx_prow, ctx_par, in_pair)


def _sc_label_gather(out_pair, bias128, lab_prow, brow_ids):
  mesh = plsc.VectorSubcoreMesh(core_axis_name="c", subcore_axis_name="s")

  @functools.partial(
      pl.kernel,
      out_type=(
          jax.ShapeDtypeStruct((B, 128), jnp.float32),
          jax.ShapeDtypeStruct((B, 128), jnp.float32),
      ),
      mesh=mesh,
      compiler_params=_SC_PARAMS,
      scratch_types=[
          pltpu.VMEM((EPW,), jnp.int32),
          pltpu.VMEM((EPW, 128), jnp.float32),
          pltpu.VMEM((EPW,), jnp.int32),
          pltpu.VMEM((EPW, 128), jnp.float32),
          pltpu.SemaphoreType.DMA,
      ],
  )
  def k(pair_h, b128_h, lab_h, rid_h, le_h, br_h, idxl, labv, ridv, brv, sem):
    wid = lax.axis_index("s") * NC + lax.axis_index("c")
    eb = wid * EPW
    pltpu.sync_copy(lab_h.at[pl.ds(eb, EPW)], idxl)
    pltpu.sync_copy(rid_h.at[pl.ds(eb, EPW)], ridv)
    h2 = pltpu.async_copy(pair_h.at[idxl], labv, sem)
    h3 = pltpu.async_copy(b128_h.at[ridv], brv, sem)
    h2.wait()
    h3.wait()
    pltpu.sync_copy(labv, le_h.at[pl.ds(eb, EPW)])
    pltpu.sync_copy(brv, br_h.at[pl.ds(eb, EPW)])

  return k(out_pair, bias128, lab_prow, brow_ids)


def _lse_body(x_ref, w_ref, be_ref, bo_ref, o_ref, xb_ref, m_ref, s_ref):
  i = pl.program_id(0)
  nv = pl.num_programs(0)

  @pl.when(i == 0)
  def _():
    xb_ref[...] = x_ref[:, 0:D].astype(jnp.bfloat16)
    m_ref[...] = jnp.full((B, 1), NEG, jnp.float32)
    s_ref[...] = jnp.zeros((B, 1), jnp.float32)

  xb = xb_ref[...]
  te = lax.dot_general(xb, w_ref[:, 0:D], (((1,), (1,)), ((), ())),
                       preferred_element_type=jnp.float32).astype(jnp.bfloat16)
  to = lax.dot_general(xb, w_ref[:, D:2 * D], (((1,), (1,)), ((), ())),
                       preferred_element_type=jnp.float32).astype(jnp.bfloat16)
  te = te + be_ref[...]
  to = to + bo_ref[...]
  tmax = jnp.maximum(
      jnp.max(te, axis=1, keepdims=True),
      jnp.max(to, axis=1, keepdims=True)).astype(jnp.float32)
  m_old = m_ref[...]
  m_new = jnp.maximum(m_old, tmax)
  mb = m_new.astype(jnp.bfloat16)
  p = (jnp.sum(jnp.exp(te - mb).astype(jnp.float32), axis=1, keepdims=True) +
       jnp.sum(jnp.exp(to - mb).astype(jnp.float32), axis=1, keepdims=True))
  s_ref[...] = s_ref[...] * jnp.exp(m_old - m_new) + p
  m_ref[...] = m_new

  @pl.when(i == nv - 1)
  def _():
    o_ref[...] = m_ref[...] + jnp.log(s_ref[...])


def _tc_lse(x_aug, w_pair, be, bo):
  return pl.pallas_call(
      _lse_body,
      grid=(NPT,),
      in_specs=[
          pl.BlockSpec((B, 128), lambda i: (0, 0)),
          pl.BlockSpec((TP, 128), lambda i: (i, 0)),
          pl.BlockSpec((1, TP), lambda i: (0, i)),
          pl.BlockSpec((1, TP), lambda i: (0, i)),
      ],
      out_specs=pl.BlockSpec((B, 1), lambda i: (0, 0)),
      out_shape=jax.ShapeDtypeStruct((B, 1), jnp.float32),
      scratch_shapes=[
          pltpu.VMEM((B, D), jnp.bfloat16),
          pltpu.VMEM((B, 1), jnp.float32),
          pltpu.VMEM((B, 1), jnp.float32),
      ],
  )(x_aug, w_pair, be, bo)


def _final_body(lse_ref, x_ref, le_ref, br_ref, ph_ref, bl_ref, o_ref):
  xa = x_ref[:, 0:D]
  sel_hi = ph_ref[...] == 1
  lv = jnp.where(sel_hi, le_ref[:, D:2 * D], le_ref[:, 0:D])
  lab_logit = jnp.sum(xa * lv, axis=1, keepdims=True)
  lsel = lax.broadcasted_iota(jnp.int32, (B, 128), 1) == bl_ref[...]
  lbias = jnp.sum(jnp.where(lsel, br_ref[...], 0.0), axis=1, keepdims=True)
  o_ref[...] = lse_ref[...] - lab_logit - lbias


def _tc_final(lse, x_aug, labv, brv, phalf2, blane2):
  return pl.pallas_call(
      _final_body,
      out_shape=jax.ShapeDtypeStruct((B, 1), jnp.float32),
  )(lse, x_aug, labv, brv, phalf2, blane2)


def kernel(input_word_ids, output_word_ids, input_layer_embeddings,
           output_layer_embeddings, output_layer_bias):
  ctx = input_word_ids.reshape(B * C)
  ctx_prow = lax.shift_right_logical(ctx, 1)
  ctx_par = lax.bitwise_and(ctx, 1)
  lab_prow = lax.shift_right_logical(output_word_ids, 1)
  lab_phalf = lax.bitwise_and(output_word_ids, 1)
  brow_ids = lax.shift_right_logical(output_word_ids, 7)
  blane = lax.bitwise_and(output_word_ids, 127)

  in_pair = input_layer_embeddings.reshape(VH, 128)
  out_pair = output_layer_embeddings.reshape(VH, 128)
  bias128 = jnp.pad(output_layer_bias, (0, 2 * HP - V)).reshape(BT, 128)
  be = jnp.pad(output_layer_bias[0::2], (0, HP - VH),
               constant_values=NEG).reshape(1, HP).astype(jnp.bfloat16)
  bo = jnp.pad(output_layer_bias[1::2], (0, HP - VH),
               constant_values=NEG).reshape(1, HP).astype(jnp.bfloat16)

  w_pair = _prep_w(out_pair)
  x_aug = _sc_ctx_gather(ctx_prow, ctx_par, in_pair)
  labv, brv = _sc_label_gather(out_pair, bias128, lab_prow, brow_ids)

  lse = _tc_lse(x_aug, w_pair, be, bo)
  loss = _tc_final(lse, x_aug, labv, brv,
                   lab_phalf.reshape(B, 1), blane.reshape(B, 1))
  return loss.reshape(B)
